# baseline (speedup 1.0000x reference)
import functools

import jax
import jax.numpy as jnp
from jax import lax
from jax.experimental import pallas as pl
from jax.experimental.pallas import tpu as pltpu
from jax.experimental.pallas import tpu_sc as plsc

K = 8192
D = 64
N = 8192
TILE_N = 256
N_TILES = N // TILE_N
N_CHUNKS = 8

NC = 2
NS = 16
NW = NC * NS
BPW = N // NW
L = 16


def _argmin_body(z_ref, w_ref, idx_ref, minval_ref, wsq_ref, wbf_ref):
    @pl.when(pl.program_id(0) == 0)
    def _():
        w = w_ref[...]
        wsq_ref[...] = jnp.sum(w * w, axis=1)[None, :]
        wbf_ref[...] = w.astype(jnp.bfloat16)

    f = z_ref[0].T
    fsq = jnp.sum(f * f, axis=1, keepdims=True)
    fb = f.astype(jnp.bfloat16)
    CK = K // N_CHUNKS
    acc_m = None
    acc_i = None
    for c in range(N_CHUNKS):
        s_c = lax.dot_general(
            fb, wbf_ref[pl.ds(c * CK, CK), :],
            (((1,), (1,)), ((), ())),
            preferred_element_type=jnp.float32)
        d_c = (fsq + wsq_ref[:, pl.ds(c * CK, CK)]) - 2.0 * s_c
        m_c = jnp.min(d_c, axis=1)
        iota = lax.broadcasted_iota(jnp.int32, d_c.shape, 1) + c * CK
        i_c = jnp.min(jnp.where(d_c == m_c[:, None], iota, jnp.int32(K)),
                      axis=1)
        if acc_m is None:
            acc_m, acc_i = m_c, i_c
        else:
            upd = m_c < acc_m
            acc_m = jnp.where(upd, m_c, acc_m)
            acc_i = jnp.where(upd, i_c, acc_i)
    idx_ref[0, 0, :] = acc_i
    minval_ref[0, 0, :] = acc_m


_argmin_call = pl.pallas_call(
    _argmin_body,
    grid=(N_TILES,),
    in_specs=[
        pl.BlockSpec((1, D, TILE_N), lambda i: (i // (1024 // TILE_N), 0,
                                                i % (1024 // TILE_N))),
        pl.BlockSpec((K, D), lambda i: (0, 0)),
    ],
    out_specs=[
        pl.BlockSpec((1, 1, TILE_N), lambda i: (i, 0, 0)),
        pl.BlockSpec((1, 1, TILE_N), lambda i: (i, 0, 0)),
    ],
    out_shape=[
        jax.ShapeDtypeStruct((N_TILES, 1, TILE_N), jnp.int32),
        jax.ShapeDtypeStruct((N_TILES, 1, TILE_N), jnp.float32),
    ],
    scratch_shapes=[
        pltpu.VMEM((1, K), jnp.float32),
        pltpu.VMEM((K, D), jnp.bfloat16),
    ],
)


def _sc_gather_body(w_hbm, idx_hbm, zeros_hbm, zq_hbm, counts_hbm,
                    idx_v, rows_v, ones_v, cnt_sh, sem):
    c = lax.axis_index("c")
    s = lax.axis_index("s")
    wid = s * NC + c
    base = wid * BPW

    @pl.when(s == 0)
    def _():
        pltpu.sync_copy(zeros_hbm, cnt_sh)

    for i in range(BPW // L):
        ones_v[pl.ds(i * L, L)] = jnp.ones((L,), jnp.float32)

    pltpu.sync_copy(idx_hbm.at[pl.ds(base, BPW)], idx_v)
    pltpu.async_copy(w_hbm.at[idx_v], rows_v, sem).wait()
    pltpu.sync_copy(rows_v, zq_hbm.at[pl.ds(base, BPW)])

    plsc.subcore_barrier()
    pltpu.sync_copy(ones_v, cnt_sh.at[idx_v], add=True)
    plsc.subcore_barrier()

    @pl.when(s == 0)
    def _():
        pltpu.sync_copy(cnt_sh, counts_hbm.at[c])


@functools.cache
def _sc_gather_call():
    return pl.kernel(
        _sc_gather_body,
        out_type=[
            jax.ShapeDtypeStruct((N, D), jnp.float32),
            jax.ShapeDtypeStruct((NC, K), jnp.float32),
        ],
        mesh=plsc.VectorSubcoreMesh(core_axis_name="c", subcore_axis_name="s",
                                    num_cores=NC, num_subcores=NS),
        scratch_types=[
            pltpu.VMEM((BPW,), jnp.int32),
            pltpu.VMEM((BPW, D), jnp.float32),
            pltpu.VMEM((BPW,), jnp.float32),
            pltpu.VMEM_SHARED((K,), jnp.float32),
            pltpu.SemaphoreType.DMA,
        ],
        compiler_params=pltpu.CompilerParams(use_tc_tiling_on_sc=False),
    )


def _finalize_body(minval_ref, counts_ref, loss_ref, perp_ref):
    mv = minval_ref[...]
    loss_ref[...] = jnp.reshape(jnp.sum(mv) * (1.25 / (N * D)), (1, 1))
    c2 = counts_ref[...]
    p = jnp.sum(c2, axis=0, keepdims=True) * (1.0 / N)
    ent = -jnp.sum(p * jnp.log(p + 1e-10))
    perp_ref[...] = jnp.reshape(jnp.exp(ent), (1, 1))


_finalize_call = pl.pallas_call(
    _finalize_body,
    out_shape=[
        jax.ShapeDtypeStruct((1, 1), jnp.float32),
        jax.ShapeDtypeStruct((1, 1), jnp.float32),
    ],
)


def kernel(z, W):
    B, C, H, Wd = z.shape
    zr = z.reshape(B, C, H * Wd)

    idx3, minval3 = _argmin_call(zr, W)
    idx = idx3.reshape(N)

    zeros = jnp.zeros((K,), jnp.float32)
    z_q, counts = _sc_gather_call()(W, idx, zeros)

    loss11, perp11 = _finalize_call(minval3.reshape(64, 128), counts)

    out = jnp.transpose(z_q.reshape(B, H, Wd, C), (0, 3, 1, 2))
    return (out, loss11[0, 0], idx.reshape(B, H, Wd), perp11[0, 0])


# speedup vs baseline: 1.1676x; 1.1676x over previous
import functools

import jax
import jax.numpy as jnp
from jax import lax
from jax.experimental import pallas as pl
from jax.experimental.pallas import tpu as pltpu
from jax.experimental.pallas import tpu_sc as plsc

K = 8192
D = 64
N = 8192
TILE_N = 256
N_TILES = N // TILE_N

NC = 2
NS = 16
NW = NC * NS
BPW = N // NW
L = 16


def _argmin_body(z_ref, w_ref, idx_ref, minval_ref, wsq_ref, wbf_ref):
    @pl.when(pl.program_id(0) == 0)
    def _():
        w = w_ref[...]
        wsq_ref[...] = jnp.sum(w * w, axis=1)[None, :]
        wbf_ref[...] = w.astype(jnp.bfloat16)

    f = z_ref[0].T
    fsq = jnp.sum(f * f, axis=1, keepdims=True)
    scores = lax.dot_general(
        f.astype(jnp.bfloat16), wbf_ref[...],
        (((1,), (1,)), ((), ())),
        preferred_element_type=jnp.float32)
    d = (fsq + wsq_ref[...]) - 2.0 * scores
    m = jnp.min(d, axis=1)
    iota = lax.broadcasted_iota(jnp.int32, d.shape, 1)
    idx_ref[0, :] = jnp.min(
        jnp.where(d == m[:, None], iota, jnp.int32(K)), axis=1)
    minval_ref[0, :] = m


_argmin_call = pl.pallas_call(
    _argmin_body,
    grid=(N_TILES,),
    in_specs=[
        pl.BlockSpec((1, D, TILE_N), lambda i: (i // (1024 // TILE_N), 0,
                                                i % (1024 // TILE_N))),
        pl.BlockSpec((K, D), lambda i: (0, 0)),
    ],
    out_specs=[
        pl.BlockSpec((1, TILE_N), lambda i: (0, i)),
        pl.BlockSpec((1, TILE_N), lambda i: (0, i)),
    ],
    out_shape=[
        jax.ShapeDtypeStruct((1, N), jnp.int32),
        jax.ShapeDtypeStruct((1, N), jnp.float32),
    ],
    scratch_shapes=[
        pltpu.VMEM((1, K), jnp.float32),
        pltpu.VMEM((K, D), jnp.bfloat16),
    ],
)


def _sc_gather_body(w_hbm, idx_hbm, zeros_hbm, zq_hbm, counts_hbm,
                    idx_v, rows_v, ones_v, cnt_sh, sem):
    c = lax.axis_index("c")
    s = lax.axis_index("s")
    wid = s * NC + c
    base = wid * BPW

    @pl.when(s == 0)
    def _():
        pltpu.sync_copy(zeros_hbm, cnt_sh)

    for i in range(BPW // L):
        ones_v[pl.ds(i * L, L)] = jnp.ones((L,), jnp.float32)

    pltpu.sync_copy(idx_hbm.at[pl.ds(base, BPW)], idx_v)
    pltpu.async_copy(w_hbm.at[idx_v], rows_v, sem).wait()
    pltpu.sync_copy(rows_v, zq_hbm.at[pl.ds(base, BPW)])

    plsc.subcore_barrier()
    pltpu.sync_copy(ones_v, cnt_sh.at[idx_v], add=True)
    plsc.subcore_barrier()

    @pl.when(s == 0)
    def _():
        pltpu.sync_copy(cnt_sh, counts_hbm.at[c])


@functools.cache
def _sc_gather_call():
    return pl.kernel(
        _sc_gather_body,
        out_type=[
            jax.ShapeDtypeStruct((N, D), jnp.float32),
            jax.ShapeDtypeStruct((NC, K), jnp.float32),
        ],
        mesh=plsc.VectorSubcoreMesh(core_axis_name="c", subcore_axis_name="s",
                                    num_cores=NC, num_subcores=NS),
        scratch_types=[
            pltpu.VMEM((BPW,), jnp.int32),
            pltpu.VMEM((BPW, D), jnp.float32),
            pltpu.VMEM((BPW,), jnp.float32),
            pltpu.VMEM_SHARED((K,), jnp.float32),
            pltpu.SemaphoreType.DMA,
        ],
        compiler_params=pltpu.CompilerParams(use_tc_tiling_on_sc=False),
    )


def _finalize_body(minval_ref, counts_ref, loss_ref, perp_ref):
    mv = minval_ref[...]
    loss_ref[...] = jnp.reshape(jnp.sum(mv) * (1.25 / (N * D)), (1, 1))
    c2 = counts_ref[...]
    p = jnp.sum(c2, axis=0, keepdims=True) * (1.0 / N)
    ent = -jnp.sum(p * jnp.log(p + 1e-10))
    perp_ref[...] = jnp.reshape(jnp.exp(ent), (1, 1))


_finalize_call = pl.pallas_call(
    _finalize_body,
    out_shape=[
        jax.ShapeDtypeStruct((1, 1), jnp.float32),
        jax.ShapeDtypeStruct((1, 1), jnp.float32),
    ],
)


def kernel(z, W):
    B, C, H, Wd = z.shape
    zr = z.reshape(B, C, H * Wd)

    idx2, minval2 = _argmin_call(zr, W)
    idx = idx2.reshape(N)

    zeros = jnp.zeros((K,), jnp.float32)
    z_q, counts = _sc_gather_call()(W, idx, zeros)

    loss11, perp11 = _finalize_call(minval2, counts)

    out = jnp.transpose(z_q.reshape(B, H, Wd, C), (0, 3, 1, 2))
    return (out, loss11[0, 0], idx.reshape(B, H, Wd), perp11[0, 0])
